# baseline (speedup 1.0000x reference)
import functools
import jax
import jax.numpy as jnp
from jax import lax
from jax.experimental import pallas as pl
from jax.experimental.pallas import tpu as pltpu
from jax.experimental.pallas import tpu_sc as plsc

T = 2048
D_MODEL = 768
D_FF = 3072
E = 8
TOPK = 2

B = 256
NTOT = T * TOPK + E * B
NB = NTOT // B

NC, NS = 2, 16
NW = NC * NS

_INTERPRET = False


_PB = 128
_PG = T // _PB


def _excl_prefix(ohb, tril):
    prefs = []
    bsums = []
    run = jnp.zeros((1, E), jnp.float32)
    for g in range(_PG):
        blk = ohb[g]
        intra = lax.dot_general(tril, blk, (((1,), (0,)), ((), ())),
                                precision=lax.Precision.HIGHEST,
                                preferred_element_type=jnp.float32)
        prefs.append(intra + run)
        run = run + jnp.sum(blk, axis=0, keepdims=True)
    return jnp.stack(prefs, axis=0), run


def _router_body(x_ref, wr_ref, q0_ref, q1_ref, g0_ref, g1_ref, meta_ref):
    x = x_ref[...]
    wr = wr_ref[...]
    logits = lax.dot_general(x, wr, (((1,), (0,)), ((), ())),
                             preferred_element_type=jnp.float32)
    m = jnp.max(logits, axis=-1, keepdims=True)
    p = jnp.exp(logits - m)
    probs = p / jnp.sum(p, axis=-1, keepdims=True)
    e_iota = lax.broadcasted_iota(jnp.int32, probs.shape, 1)
    v1 = jnp.max(probs, axis=-1, keepdims=True)
    i1 = jnp.min(jnp.where(probs == v1, e_iota, E), axis=-1, keepdims=True)
    masked = jnp.where(e_iota == i1, -1.0, probs)
    v2 = jnp.max(masked, axis=-1, keepdims=True)
    i2 = jnp.min(jnp.where(masked == v2, e_iota, E), axis=-1, keepdims=True)
    denom = v1 + v2 + 1e-9
    g0_ref[...] = v1 / denom
    g1_ref[...] = v2 / denom

    oh1 = jnp.where(i1 == e_iota, 1.0, 0.0)
    oh2 = jnp.where(i2 == e_iota, 1.0, 0.0)
    r_iota = lax.broadcasted_iota(jnp.int32, (_PB, _PB), 0)
    c_iota = lax.broadcasted_iota(jnp.int32, (_PB, _PB), 1)
    tril = jnp.where(r_iota > c_iota, 1.0, 0.0)
    pref1, cnt1 = _excl_prefix(oh1.reshape(_PG, _PB, E), tril)
    pref2, cnt2 = _excl_prefix(oh2.reshape(_PG, _PB, E), tril)
    rank1 = pref1.reshape(T, E)
    rank2 = pref2.reshape(T, E) + cnt1
    cnt = cnt1 + cnt2
    pc = jnp.floor((cnt + (B - 1)) * (1.0 / B)) * B
    run = jnp.zeros((1, 1), jnp.float32)
    cums = []
    for e in range(E):
        run = run + pc[:, e:e + 1]
        cums.append(run)
    cum = jnp.concatenate(cums, axis=1)
    poff = cum - pc
    q0f = jnp.sum(oh1 * (poff + rank1), axis=1, keepdims=True)
    q1f = jnp.sum(oh2 * (poff + rank2), axis=1, keepdims=True)
    q0_ref[...] = q0f.astype(jnp.int32)
    q1_ref[...] = q1f.astype(jnp.int32)

    bstart = lax.broadcasted_iota(jnp.int32, (32, E), 0).astype(jnp.float32) * B
    be = jnp.sum(jnp.where(cum <= bstart, 1.0, 0.0), axis=1, keepdims=True)
    be = jnp.minimum(be, E - 1)
    nactive = cum[:, E - 1:] * (1.0 / B)
    m_iota = lax.broadcasted_iota(jnp.int32, (32, 1), 0)
    meta = jnp.where(m_iota < NB, be,
                     jnp.where(m_iota == NB, nactive, 0.0))
    meta_ref[...] = meta.astype(jnp.int32)


def _dispatch_body(x_hbm, q0_hbm, q1_hbm, g0_hbm, g1_hbm, xg_hbm, gs_hbm,
                   rows_v, q0_v, q1_v, g0_v, g1_v, sem):
    wid = lax.axis_index("s") * NC + lax.axis_index("c")
    base = wid * (T // NW)
    pltpu.sync_copy(x_hbm.at[pl.ds(base, 64)], rows_v)
    pltpu.sync_copy(q0_hbm.at[pl.ds(base, 64)], q0_v)
    pltpu.sync_copy(q1_hbm.at[pl.ds(base, 64)], q1_v)
    pltpu.sync_copy(g0_hbm.at[pl.ds(base, 64)], g0_v)
    pltpu.sync_copy(g1_hbm.at[pl.ds(base, 64)], g1_v)
    cp0 = pltpu.async_copy(rows_v, xg_hbm.at[q0_v], sem)
    cp1 = pltpu.async_copy(rows_v, xg_hbm.at[q1_v], sem)
    cp2 = pltpu.async_copy(g0_v, gs_hbm.at[q0_v], sem)
    cp3 = pltpu.async_copy(g1_v, gs_hbm.at[q1_v], sem)
    cp0.wait()
    cp1.wait()
    cp2.wait()
    cp3.wait()


def _ffn_body(meta_ref, xg_ref, gs_ref, w1_ref, b1_ref, w2_ref, b2_ref,
              out_ref):
    b = pl.program_id(0)

    @pl.when(b < meta_ref[NB])
    def _():
        x = xg_ref[...].astype(jnp.bfloat16)
        w1 = w1_ref[0]
        w2 = w2_ref[0]
        h = lax.dot_general(x, w1, (((1,), (0,)), ((), ())),
                            preferred_element_type=jnp.float32) + b1_ref[0]
        h = jax.nn.gelu(h).astype(jnp.bfloat16)
        y = lax.dot_general(h, w2, (((1,), (0,)), ((), ())),
                            preferred_element_type=jnp.float32) + b2_ref[0]
        out_ref[...] = y * gs_ref[...]


def _combine_body(y_hbm, q0_hbm, q1_hbm, out_hbm,
                  r0_v, r1_v, q0_v, q1_v, sem0, sem1):
    wid = lax.axis_index("s") * NC + lax.axis_index("c")
    tpw = T // NW
    CH = 32

    def chunk(i, _):
        base = wid * tpw + i * CH
        pltpu.sync_copy(q0_hbm.at[pl.ds(base, CH)], q0_v)
        pltpu.sync_copy(q1_hbm.at[pl.ds(base, CH)], q1_v)
        cp0 = pltpu.async_copy(y_hbm.at[q0_v], r0_v, sem0)
        cp1 = pltpu.async_copy(y_hbm.at[q1_v], r1_v, sem1)
        cp0.wait()
        cp1.wait()

        def row(j, _):
            for c in range(D_MODEL // 16):
                sl = pl.ds(c * 16, 16)
                r0_v[j, sl] = r0_v[j, sl] + r1_v[j, sl]
            return 0

        lax.fori_loop(0, CH, row, 0)
        pltpu.sync_copy(r0_v, out_hbm.at[pl.ds(base, CH)])
        return 0

    lax.fori_loop(0, tpw // CH, chunk, 0)


def kernel(x, Wr, W1, b1, W2, b2):
    q0c, q1c, g0c, g1c, metac = pl.pallas_call(
        _router_body,
        out_shape=(jax.ShapeDtypeStruct((T, 1), jnp.int32),
                   jax.ShapeDtypeStruct((T, 1), jnp.int32),
                   jax.ShapeDtypeStruct((T, 1), jnp.float32),
                   jax.ShapeDtypeStruct((T, 1), jnp.float32),
                   jax.ShapeDtypeStruct((32, 1), jnp.int32)),
        interpret=_INTERPRET,
    )(x, Wr)
    q0 = q0c.reshape(T)
    q1 = q1c.reshape(T)
    meta = metac.reshape(32)[:NB + 1]
    W1b = W1.astype(jnp.bfloat16)
    W2b = W2.astype(jnp.bfloat16)

    mesh = plsc.VectorSubcoreMesh(core_axis_name="c", subcore_axis_name="s")

    xg, gs = pl.kernel(
        _dispatch_body,
        out_type=(jax.ShapeDtypeStruct((NTOT, D_MODEL), jnp.float32),
                  jax.ShapeDtypeStruct((NTOT,), jnp.float32)),
        mesh=mesh,
        scratch_types=[
            pltpu.VMEM((64, D_MODEL), jnp.float32),
            pltpu.VMEM((64,), jnp.int32),
            pltpu.VMEM((64,), jnp.int32),
            pltpu.VMEM((64,), jnp.float32),
            pltpu.VMEM((64,), jnp.float32),
            pltpu.SemaphoreType.DMA,
        ],
    )(x, q0, q1, g0c.reshape(T), g1c.reshape(T))

    y = pl.pallas_call(
        _ffn_body,
        grid_spec=pltpu.PrefetchScalarGridSpec(
            num_scalar_prefetch=1,
            grid=(NB,),
            in_specs=[
                pl.BlockSpec((B, D_MODEL), lambda b, m: (b, 0)),
                pl.BlockSpec((B, 1), lambda b, m: (b, 0)),
                pl.BlockSpec((1, D_MODEL, D_FF), lambda b, m: (m[b], 0, 0)),
                pl.BlockSpec((1, 1, D_FF), lambda b, m: (m[b], 0, 0)),
                pl.BlockSpec((1, D_FF, D_MODEL), lambda b, m: (m[b], 0, 0)),
                pl.BlockSpec((1, 1, D_MODEL), lambda b, m: (m[b], 0, 0)),
            ],
            out_specs=pl.BlockSpec((B, D_MODEL), lambda b, m: (b, 0)),
        ),
        out_shape=jax.ShapeDtypeStruct((NTOT, D_MODEL), jnp.float32),
        compiler_params=pltpu.CompilerParams(
            dimension_semantics=("arbitrary",),
        ),
        interpret=_INTERPRET,
    )(meta, xg, gs.reshape(NTOT, 1), W1b, b1.reshape(E, 1, D_FF), W2b,
      b2.reshape(E, 1, D_MODEL))

    out = pl.kernel(
        _combine_body,
        out_type=jax.ShapeDtypeStruct((T, D_MODEL), jnp.float32),
        mesh=mesh,
        scratch_types=[
            pltpu.VMEM((32, D_MODEL), jnp.float32),
            pltpu.VMEM((32, D_MODEL), jnp.float32),
            pltpu.VMEM((32,), jnp.int32),
            pltpu.VMEM((32,), jnp.int32),
            pltpu.SemaphoreType.DMA,
            pltpu.SemaphoreType.DMA,
        ],
    )(y, q0, q1)

    return out


# speedup vs baseline: 1.3996x; 1.3996x over previous
import functools
import jax
import jax.numpy as jnp
from jax import lax
from jax.experimental import pallas as pl
from jax.experimental.pallas import tpu as pltpu
from jax.experimental.pallas import tpu_sc as plsc

T = 2048
D_MODEL = 768
D_FF = 3072
E = 8
TOPK = 2

B = 256
NTOT = T * TOPK + E * B
NB = NTOT // B

NC, NS = 2, 16
NW = NC * NS

_INTERPRET = False


_PB = 128
_PG = T // _PB


def _excl_prefix(ohb, tril):
    prefs = []
    bsums = []
    run = jnp.zeros((1, E), jnp.float32)
    for g in range(_PG):
        blk = ohb[g]
        intra = lax.dot_general(tril, blk, (((1,), (0,)), ((), ())),
                                precision=lax.Precision.HIGHEST,
                                preferred_element_type=jnp.float32)
        prefs.append(intra + run)
        run = run + jnp.sum(blk, axis=0, keepdims=True)
    return jnp.stack(prefs, axis=0), run


def _router_body(x_ref, wr_ref, q0_ref, q1_ref, xp0_ref, xp1_ref, meta_ref):
    x = x_ref[...]
    wr = wr_ref[...]
    logits = lax.dot_general(x, wr, (((1,), (0,)), ((), ())),
                             preferred_element_type=jnp.float32)
    m = jnp.max(logits, axis=-1, keepdims=True)
    p = jnp.exp(logits - m)
    probs = p / jnp.sum(p, axis=-1, keepdims=True)
    e_iota = lax.broadcasted_iota(jnp.int32, probs.shape, 1)
    v1 = jnp.max(probs, axis=-1, keepdims=True)
    i1 = jnp.min(jnp.where(probs == v1, e_iota, E), axis=-1, keepdims=True)
    masked = jnp.where(e_iota == i1, -1.0, probs)
    v2 = jnp.max(masked, axis=-1, keepdims=True)
    i2 = jnp.min(jnp.where(masked == v2, e_iota, E), axis=-1, keepdims=True)
    denom = v1 + v2 + 1e-9
    l_iota = lax.broadcasted_iota(jnp.int32, (T, 128), 1)
    xp0_ref[...] = jnp.concatenate(
        [x, jnp.where(l_iota == 0, v1 / denom, 0.0)], axis=1)
    xp1_ref[...] = jnp.concatenate(
        [x, jnp.where(l_iota == 0, v2 / denom, 0.0)], axis=1)

    oh1 = jnp.where(i1 == e_iota, 1.0, 0.0)
    oh2 = jnp.where(i2 == e_iota, 1.0, 0.0)
    r_iota = lax.broadcasted_iota(jnp.int32, (_PB, _PB), 0)
    c_iota = lax.broadcasted_iota(jnp.int32, (_PB, _PB), 1)
    tril = jnp.where(r_iota > c_iota, 1.0, 0.0)
    pref1, cnt1 = _excl_prefix(oh1.reshape(_PG, _PB, E), tril)
    pref2, cnt2 = _excl_prefix(oh2.reshape(_PG, _PB, E), tril)
    rank1 = pref1.reshape(T, E)
    rank2 = pref2.reshape(T, E) + cnt1
    cnt = cnt1 + cnt2
    pc = jnp.floor((cnt + (B - 1)) * (1.0 / B)) * B
    run = jnp.zeros((1, 1), jnp.float32)
    cums = []
    for e in range(E):
        run = run + pc[:, e:e + 1]
        cums.append(run)
    cum = jnp.concatenate(cums, axis=1)
    poff = cum - pc
    q0f = jnp.sum(oh1 * (poff + rank1), axis=1, keepdims=True)
    q1f = jnp.sum(oh2 * (poff + rank2), axis=1, keepdims=True)
    q0_ref[...] = q0f.astype(jnp.int32)
    q1_ref[...] = q1f.astype(jnp.int32)

    bstart = lax.broadcasted_iota(jnp.int32, (32, E), 0).astype(jnp.float32) * B
    be = jnp.sum(jnp.where(cum <= bstart, 1.0, 0.0), axis=1, keepdims=True)
    be = jnp.minimum(be, E - 1)
    nactive = cum[:, E - 1:] * (1.0 / B)
    m_iota = lax.broadcasted_iota(jnp.int32, (32, 1), 0)
    meta = jnp.where(m_iota < NB, be,
                     jnp.where(m_iota == NB, nactive, 0.0))
    meta_ref[...] = meta.astype(jnp.int32)


def _dispatch_body(xp0_hbm, xp1_hbm, q0_hbm, q1_hbm, xg_hbm,
                   rows0_v, rows1_v, q0_v, q1_v, sem):
    wid = lax.axis_index("s") * NC + lax.axis_index("c")
    base = wid * (T // NW)
    pltpu.sync_copy(xp0_hbm.at[pl.ds(base, 64)], rows0_v)
    pltpu.sync_copy(xp1_hbm.at[pl.ds(base, 64)], rows1_v)
    pltpu.sync_copy(q0_hbm.at[pl.ds(base, 64)], q0_v)
    pltpu.sync_copy(q1_hbm.at[pl.ds(base, 64)], q1_v)
    cp0 = pltpu.async_copy(rows0_v, xg_hbm.at[q0_v], sem)
    cp1 = pltpu.async_copy(rows1_v, xg_hbm.at[q1_v], sem)
    cp0.wait()
    cp1.wait()


def _ffn_body(meta_ref, xg_ref, w1_ref, b1_ref, w2_ref, b2_ref,
              out_ref):
    b = pl.program_id(0)

    @pl.when(b < meta_ref[NB])
    def _():
        x = xg_ref[...][:, :D_MODEL].astype(jnp.bfloat16)
        w1 = w1_ref[0].astype(jnp.bfloat16)
        w2 = w2_ref[0].astype(jnp.bfloat16)
        h = lax.dot_general(x, w1, (((1,), (0,)), ((), ())),
                            preferred_element_type=jnp.float32) + b1_ref[0]
        h = jax.nn.gelu(h).astype(jnp.bfloat16)
        y = lax.dot_general(h, w2, (((1,), (0,)), ((), ())),
                            preferred_element_type=jnp.float32) + b2_ref[0]
        out_ref[...] = y * xg_ref[...][:, D_MODEL:D_MODEL + 1]


def _combine_body(y_hbm, q0_hbm, q1_hbm, out_hbm,
                  r0_v, r1_v, q0_v, q1_v, sem0, sem1):
    wid = lax.axis_index("s") * NC + lax.axis_index("c")
    tpw = T // NW
    CH = 32

    def chunk(i, _):
        base = wid * tpw + i * CH
        pltpu.sync_copy(q0_hbm.at[pl.ds(base, CH)], q0_v)
        pltpu.sync_copy(q1_hbm.at[pl.ds(base, CH)], q1_v)
        cp0 = pltpu.async_copy(y_hbm.at[q0_v], r0_v, sem0)
        cp1 = pltpu.async_copy(y_hbm.at[q1_v], r1_v, sem1)
        cp0.wait()
        cp1.wait()

        def row(j, _):
            for c in range(D_MODEL // 16):
                sl = pl.ds(c * 16, 16)
                r0_v[j, sl] = r0_v[j, sl] + r1_v[j, sl]
            return 0

        lax.fori_loop(0, CH, row, 0)
        pltpu.sync_copy(r0_v, out_hbm.at[pl.ds(base, CH)])
        return 0

    lax.fori_loop(0, tpw // CH, chunk, 0)


def kernel(x, Wr, W1, b1, W2, b2):
    q0c, q1c, xp0, xp1, metac = pl.pallas_call(
        _router_body,
        out_shape=(jax.ShapeDtypeStruct((T, 1), jnp.int32),
                   jax.ShapeDtypeStruct((T, 1), jnp.int32),
                   jax.ShapeDtypeStruct((T, D_MODEL + 128), jnp.float32),
                   jax.ShapeDtypeStruct((T, D_MODEL + 128), jnp.float32),
                   jax.ShapeDtypeStruct((32, 1), jnp.int32)),
        interpret=_INTERPRET,
    )(x, Wr)
    q0 = q0c.reshape(T)
    q1 = q1c.reshape(T)
    meta = metac.reshape(32)[:NB + 1]

    mesh = plsc.VectorSubcoreMesh(core_axis_name="c", subcore_axis_name="s")

    xg = pl.kernel(
        _dispatch_body,
        out_type=jax.ShapeDtypeStruct((NTOT, D_MODEL + 128), jnp.float32),
        mesh=mesh,
        scratch_types=[
            pltpu.VMEM((64, D_MODEL + 128), jnp.float32),
            pltpu.VMEM((64, D_MODEL + 128), jnp.float32),
            pltpu.VMEM((64,), jnp.int32),
            pltpu.VMEM((64,), jnp.int32),
            pltpu.SemaphoreType.DMA,
        ],
    )(xp0, xp1, q0, q1)

    y = pl.pallas_call(
        _ffn_body,
        grid_spec=pltpu.PrefetchScalarGridSpec(
            num_scalar_prefetch=1,
            grid=(NB,),
            in_specs=[
                pl.BlockSpec((B, D_MODEL + 128), lambda b, m: (b, 0)),
                pl.BlockSpec((1, D_MODEL, D_FF), lambda b, m: (m[b], 0, 0)),
                pl.BlockSpec((1, 1, D_FF), lambda b, m: (m[b], 0, 0)),
                pl.BlockSpec((1, D_FF, D_MODEL), lambda b, m: (m[b], 0, 0)),
                pl.BlockSpec((1, 1, D_MODEL), lambda b, m: (m[b], 0, 0)),
            ],
            out_specs=pl.BlockSpec((B, D_MODEL), lambda b, m: (b, 0)),
        ),
        out_shape=jax.ShapeDtypeStruct((NTOT, D_MODEL), jnp.float32),
        compiler_params=pltpu.CompilerParams(
            dimension_semantics=("arbitrary",),
        ),
        interpret=_INTERPRET,
    )(meta, xg, W1, b1.reshape(E, 1, D_FF), W2,
      b2.reshape(E, 1, D_MODEL))

    out = pl.kernel(
        _combine_body,
        out_type=jax.ShapeDtypeStruct((T, D_MODEL), jnp.float32),
        mesh=mesh,
        scratch_types=[
            pltpu.VMEM((32, D_MODEL), jnp.float32),
            pltpu.VMEM((32, D_MODEL), jnp.float32),
            pltpu.VMEM((32,), jnp.int32),
            pltpu.VMEM((32,), jnp.int32),
            pltpu.SemaphoreType.DMA,
            pltpu.SemaphoreType.DMA,
        ],
    )(y, q0, q1)

    return out


# speedup vs baseline: 1.4107x; 1.0080x over previous
import jax
import jax.numpy as jnp
from jax import lax
from jax.experimental import pallas as pl
from jax.experimental.pallas import tpu as pltpu
from jax.experimental.pallas import tpu_sc as plsc

T = 2048
D_MODEL = 768
D_FF = 3072
E = 8
TOPK = 2

B = 256
NTOT = T * TOPK + E * B
NB = NTOT // B

NC, NS = 2, 16
NW = NC * NS


_PB = 128
_PG = T // _PB


def _excl_prefix(ohb, tril):
    prefs = []
    bsums = []
    run = jnp.zeros((1, E), jnp.float32)
    for g in range(_PG):
        blk = ohb[g]
        intra = lax.dot_general(tril, blk, (((1,), (0,)), ((), ())),
                                preferred_element_type=jnp.float32)
        prefs.append(intra + run)
        run = run + jnp.sum(blk.astype(jnp.float32), axis=0, keepdims=True)
    return jnp.stack(prefs, axis=0), run


def _router_body(x_ref, wr_ref, q0_ref, q1_ref, xp0_ref, xp1_ref, meta_ref):
    x = x_ref[...]
    wr = wr_ref[...]
    logits = lax.dot_general(x.astype(jnp.bfloat16), wr.astype(jnp.bfloat16),
                             (((1,), (0,)), ((), ())),
                             preferred_element_type=jnp.float32)
    m = jnp.max(logits, axis=-1, keepdims=True)
    p = jnp.exp(logits - m)
    probs = p / jnp.sum(p, axis=-1, keepdims=True)
    e_iota = lax.broadcasted_iota(jnp.int32, probs.shape, 1)
    v1 = jnp.max(probs, axis=-1, keepdims=True)
    i1 = jnp.min(jnp.where(probs == v1, e_iota, E), axis=-1, keepdims=True)
    masked = jnp.where(e_iota == i1, -1.0, probs)
    v2 = jnp.max(masked, axis=-1, keepdims=True)
    i2 = jnp.min(jnp.where(masked == v2, e_iota, E), axis=-1, keepdims=True)
    denom = v1 + v2 + 1e-9
    l_iota = lax.broadcasted_iota(jnp.int32, (T, 128), 1)
    xp0_ref[...] = jnp.concatenate(
        [x, jnp.where(l_iota == 0, v1 / denom, 0.0)], axis=1)
    xp1_ref[...] = jnp.concatenate(
        [x, jnp.where(l_iota == 0, v2 / denom, 0.0)], axis=1)

    oh1 = jnp.where(i1 == e_iota, 1.0, 0.0)
    oh2 = jnp.where(i2 == e_iota, 1.0, 0.0)
    oh1b = oh1.astype(jnp.bfloat16)
    oh2b = oh2.astype(jnp.bfloat16)
    r_iota = lax.broadcasted_iota(jnp.int32, (_PB, _PB), 0)
    c_iota = lax.broadcasted_iota(jnp.int32, (_PB, _PB), 1)
    tril = jnp.where(r_iota > c_iota, 1.0, 0.0).astype(jnp.bfloat16)
    pref1, cnt1 = _excl_prefix(oh1b.reshape(_PG, _PB, E), tril)
    pref2, cnt2 = _excl_prefix(oh2b.reshape(_PG, _PB, E), tril)
    rank1 = pref1.reshape(T, E)
    rank2 = pref2.reshape(T, E) + cnt1
    cnt = cnt1 + cnt2
    pc = jnp.floor((cnt + (B - 1)) * (1.0 / B)) * B
    run = jnp.zeros((1, 1), jnp.float32)
    cums = []
    for e in range(E):
        run = run + pc[:, e:e + 1]
        cums.append(run)
    cum = jnp.concatenate(cums, axis=1)
    poff = cum - pc
    q0f = jnp.sum(oh1 * (poff + rank1), axis=1, keepdims=True)
    q1f = jnp.sum(oh2 * (poff + rank2), axis=1, keepdims=True)
    q0_ref[...] = q0f.astype(jnp.int32)
    q1_ref[...] = q1f.astype(jnp.int32)

    bstart = lax.broadcasted_iota(jnp.int32, (32, E), 0).astype(jnp.float32) * B
    be = jnp.sum(jnp.where(cum <= bstart, 1.0, 0.0), axis=1, keepdims=True)
    be = jnp.minimum(be, E - 1)
    nactive = cum[:, E - 1:] * (1.0 / B)
    m_iota = lax.broadcasted_iota(jnp.int32, (32, 1), 0)
    meta = jnp.where(m_iota < NB, be,
                     jnp.where(m_iota == NB, nactive, 0.0))
    meta_ref[...] = meta.astype(jnp.int32)


def _dispatch_body(xp0_hbm, xp1_hbm, q0_hbm, q1_hbm, xg_hbm,
                   rows0_v, rows1_v, q0_v, q1_v, sem):
    wid = lax.axis_index("s") * NC + lax.axis_index("c")
    base = wid * (T // NW)
    pltpu.sync_copy(xp0_hbm.at[pl.ds(base, 64)], rows0_v)
    pltpu.sync_copy(xp1_hbm.at[pl.ds(base, 64)], rows1_v)
    pltpu.sync_copy(q0_hbm.at[pl.ds(base, 64)], q0_v)
    pltpu.sync_copy(q1_hbm.at[pl.ds(base, 64)], q1_v)
    cp0 = pltpu.async_copy(rows0_v, xg_hbm.at[q0_v], sem)
    cp1 = pltpu.async_copy(rows1_v, xg_hbm.at[q1_v], sem)
    cp0.wait()
    cp1.wait()


def _ffn_body(meta_ref, xg_ref, w1_ref, b1_ref, w2_ref, b2_ref,
              out_ref):
    b = pl.program_id(0)

    @pl.when(b < meta_ref[NB])
    def _():
        x = xg_ref[...][:, :D_MODEL].astype(jnp.bfloat16)
        w1 = w1_ref[0].astype(jnp.bfloat16)
        w2 = w2_ref[0].astype(jnp.bfloat16)
        h = lax.dot_general(x, w1, (((1,), (0,)), ((), ())),
                            preferred_element_type=jnp.float32) + b1_ref[0]
        h = jax.nn.gelu(h).astype(jnp.bfloat16)
        y = lax.dot_general(h, w2, (((1,), (0,)), ((), ())),
                            preferred_element_type=jnp.float32) + b2_ref[0]
        out_ref[...] = y * xg_ref[...][:, D_MODEL:D_MODEL + 1]


def _combine_body(y_hbm, q0_hbm, q1_hbm, out_hbm,
                  r0_v, r1_v, q0_v, q1_v, sem0, sem1):
    wid = lax.axis_index("s") * NC + lax.axis_index("c")
    tpw = T // NW
    CH = 32

    def chunk(i, _):
        base = wid * tpw + i * CH
        pltpu.sync_copy(q0_hbm.at[pl.ds(base, CH)], q0_v)
        pltpu.sync_copy(q1_hbm.at[pl.ds(base, CH)], q1_v)
        cp0 = pltpu.async_copy(y_hbm.at[q0_v], r0_v, sem0)
        cp1 = pltpu.async_copy(y_hbm.at[q1_v], r1_v, sem1)
        cp0.wait()
        cp1.wait()

        def row(j, _):
            for c in range(D_MODEL // 16):
                sl = pl.ds(c * 16, 16)
                r0_v[j, sl] = r0_v[j, sl] + r1_v[j, sl]
            return 0

        lax.fori_loop(0, CH, row, 0)
        pltpu.sync_copy(r0_v, out_hbm.at[pl.ds(base, CH)])
        return 0

    lax.fori_loop(0, tpw // CH, chunk, 0)


def kernel(x, Wr, W1, b1, W2, b2):
    q0c, q1c, xp0, xp1, metac = pl.pallas_call(
        _router_body,
        out_shape=(jax.ShapeDtypeStruct((T, 1), jnp.int32),
                   jax.ShapeDtypeStruct((T, 1), jnp.int32),
                   jax.ShapeDtypeStruct((T, D_MODEL + 128), jnp.float32),
                   jax.ShapeDtypeStruct((T, D_MODEL + 128), jnp.float32),
                   jax.ShapeDtypeStruct((32, 1), jnp.int32)),
    )(x, Wr)
    q0 = q0c.reshape(T)
    q1 = q1c.reshape(T)
    meta = metac.reshape(32)[:NB + 1]

    mesh = plsc.VectorSubcoreMesh(core_axis_name="c", subcore_axis_name="s")

    xg = pl.kernel(
        _dispatch_body,
        out_type=jax.ShapeDtypeStruct((NTOT, D_MODEL + 128), jnp.float32),
        mesh=mesh,
        scratch_types=[
            pltpu.VMEM((64, D_MODEL + 128), jnp.float32),
            pltpu.VMEM((64, D_MODEL + 128), jnp.float32),
            pltpu.VMEM((64,), jnp.int32),
            pltpu.VMEM((64,), jnp.int32),
            pltpu.SemaphoreType.DMA,
        ],
    )(xp0, xp1, q0, q1)

    y = pl.pallas_call(
        _ffn_body,
        grid_spec=pltpu.PrefetchScalarGridSpec(
            num_scalar_prefetch=1,
            grid=(NB,),
            in_specs=[
                pl.BlockSpec((B, D_MODEL + 128), lambda b, m: (b, 0)),
                pl.BlockSpec((1, D_MODEL, D_FF), lambda b, m: (m[b], 0, 0)),
                pl.BlockSpec((1, 1, D_FF), lambda b, m: (m[b], 0, 0)),
                pl.BlockSpec((1, D_FF, D_MODEL), lambda b, m: (m[b], 0, 0)),
                pl.BlockSpec((1, 1, D_MODEL), lambda b, m: (m[b], 0, 0)),
            ],
            out_specs=pl.BlockSpec((B, D_MODEL), lambda b, m: (b, 0)),
        ),
        out_shape=jax.ShapeDtypeStruct((NTOT, D_MODEL), jnp.float32),
        compiler_params=pltpu.CompilerParams(
            dimension_semantics=("arbitrary",),
        ),
    )(meta, xg, W1, b1.reshape(E, 1, D_FF), W2,
      b2.reshape(E, 1, D_MODEL))

    out = pl.kernel(
        _combine_body,
        out_type=jax.ShapeDtypeStruct((T, D_MODEL), jnp.float32),
        mesh=mesh,
        scratch_types=[
            pltpu.VMEM((32, D_MODEL), jnp.float32),
            pltpu.VMEM((32, D_MODEL), jnp.float32),
            pltpu.VMEM((32,), jnp.int32),
            pltpu.VMEM((32,), jnp.int32),
            pltpu.SemaphoreType.DMA,
            pltpu.SemaphoreType.DMA,
        ],
    )(y, q0, q1)

    return out
